# Initial kernel scaffold; baseline (speedup 1.0000x reference)
import jax
import jax.numpy as jnp
from jax.experimental import pallas as pl


def kernel(h_V, h_E, E_idx, mask_V, mask_attend, params):
    raise NotImplementedError("write your pallas kernel here")


# speedup vs baseline: 15.6482x; 15.6482x over previous
import functools

import jax
import jax.numpy as jnp
from jax import lax
from jax.experimental import pallas as pl
from jax.experimental.pallas import tpu as pltpu
from jax.experimental.pallas import tpu_sc as plsc

B, L, K, H = 8, 2048, 32, 128
SCALE = 30.0
R = 128
E_BLK = R * K
TOTAL = B * L * K

HP = H // 2
NC, NS = 2, 16
NW = NC * NS
PER_W = TOTAL // NW
CH = 128
N_CHUNKS = PER_W // CH


def _gelu(x):
    return 0.5 * x * (1.0 + lax.erf(x * (2.0 ** -0.5)))


def _ln(x, g, b, eps=1e-5):
    m = jnp.mean(x, axis=-1, keepdims=True)
    c = x - m
    v = jnp.mean(c * c, axis=-1, keepdims=True)
    return c * lax.rsqrt(v + eps) * g + b


def _sc_gather_body(table_hbm, idx_hbm, out_hbm, idx_v, rows_v, sem):
    wid = lax.axis_index("s") * NC + lax.axis_index("c")
    base = wid * PER_W

    def body(c, carry):
        off = base + c * CH
        pltpu.sync_copy(idx_hbm.at[pl.ds(off, CH)], idx_v)
        pltpu.async_copy(table_hbm.at[idx_v], rows_v, sem).wait()
        pltpu.sync_copy(rows_v, out_hbm.at[pl.ds(off, CH)])
        return carry

    lax.fori_loop(0, N_CHUNKS, body, 0)


def _sc_gather(table, idx_flat):
    mesh = plsc.VectorSubcoreMesh(core_axis_name="c", subcore_axis_name="s",
                                  num_cores=NC, num_subcores=NS)
    out = pl.kernel(
        _sc_gather_body,
        out_type=jax.ShapeDtypeStruct((TOTAL, H), jnp.float32),
        mesh=mesh,
        scratch_types=[
            pltpu.VMEM((CH,), jnp.int32),
            pltpu.VMEM((CH, H), jnp.float32),
            pltpu.SemaphoreType.DMA,
        ],
        name="sc_neighbor_gather",
    )(table, idx_flat)
    return out.reshape(B, L * K, H)


def _node_body(hv_ref, he_ref, g1_ref, ma_ref, mv_ref,
               w1s_ref, w1en_ref, b1_ref, w2_ref, b2_ref, w3_ref, b3_ref,
               n1g_ref, n1b_ref, win_ref, bin_ref, wout_ref, bout_ref,
               n2g_ref, n2b_ref,
               out_ref, outb_ref):
    f32 = jnp.float32
    hv = hv_ref[0]
    hvb = hv.astype(jnp.bfloat16)
    pre_s = jnp.dot(hvb, w1s_ref[...], preferred_element_type=f32) + b1_ref[...]
    he = he_ref[0].astype(jnp.bfloat16)
    g1 = g1_ref[0].astype(jnp.bfloat16)
    x = jnp.concatenate([he, g1], axis=1)
    t = jnp.dot(x, w1en_ref[...], preferred_element_type=f32)
    t = t.reshape(R, K, H) + pre_s[:, None, :]
    t = _gelu(t).reshape(E_BLK, H).astype(jnp.bfloat16)
    t = jnp.dot(t, w2_ref[...], preferred_element_type=f32) + b2_ref[...]
    t = _gelu(t).astype(jnp.bfloat16)
    msg = jnp.dot(t, w3_ref[...], preferred_element_type=f32) + b3_ref[...]
    msg = msg.reshape(R, K, H) * ma_ref[0][:, :, None]
    dh = jnp.sum(msg, axis=1) * (1.0 / SCALE)
    h1 = _ln(hv + dh, n1g_ref[...], n1b_ref[...])
    ff = jnp.dot(h1.astype(jnp.bfloat16), win_ref[...],
                 preferred_element_type=f32) + bin_ref[...]
    ff = _gelu(ff).astype(jnp.bfloat16)
    d2 = jnp.dot(ff, wout_ref[...], preferred_element_type=f32) + bout_ref[...]
    h2 = _ln(h1 + d2, n2g_ref[...], n2b_ref[...]) * mv_ref[0]
    out_ref[0] = h2
    outb_ref[0] = h2.astype(jnp.bfloat16)


def _node_update(h_V, h_E2, g1, mask_attend, mask_V3, wp):
    grid = (B, L // R)
    full = lambda shape: pl.BlockSpec(shape, lambda b, i: (0,) * len(shape))
    in_specs = [
        pl.BlockSpec((1, R, H), lambda b, i: (b, i, 0)),
        pl.BlockSpec((1, E_BLK, H), lambda b, i: (b, i, 0)),
        pl.BlockSpec((1, E_BLK, H), lambda b, i: (b, i, 0)),
        pl.BlockSpec((1, R, K), lambda b, i: (b, i, 0)),
        pl.BlockSpec((1, R, 1), lambda b, i: (b, i, 0)),
        full((H, H)), full((2 * H, H)), full((1, H)),
        full((H, H)), full((1, H)), full((H, H)), full((1, H)),
        full((1, H)), full((1, H)),
        full((H, 4 * H)), full((1, 4 * H)), full((4 * H, H)), full((1, H)),
        full((1, H)), full((1, H)),
    ]
    out_specs = [
        pl.BlockSpec((1, R, H), lambda b, i: (b, i, 0)),
        pl.BlockSpec((1, R, H), lambda b, i: (b, i, 0)),
    ]
    return pl.pallas_call(
        _node_body,
        grid=grid,
        in_specs=in_specs,
        out_specs=out_specs,
        out_shape=[
            jax.ShapeDtypeStruct((B, L, H), jnp.float32),
            jax.ShapeDtypeStruct((B, L, H), jnp.bfloat16),
        ],
        name="tc_node_update",
    )(h_V, h_E2, g1, mask_attend, mask_V3, *wp)


def _edge_body(he_ref, g2_ref, hv2_ref,
               w1s_ref, w1en_ref, b1_ref, w2_ref, b2_ref, w3_ref, b3_ref,
               n3g_ref, n3b_ref, out_ref):
    f32 = jnp.float32
    hv2 = hv2_ref[0]
    pre_s = jnp.dot(hv2, w1s_ref[...], preferred_element_type=f32) + b1_ref[...]
    he = he_ref[0]
    x = jnp.concatenate([he.astype(jnp.bfloat16),
                         g2_ref[0].astype(jnp.bfloat16)], axis=1)
    t = jnp.dot(x, w1en_ref[...], preferred_element_type=f32)
    t = t.reshape(R, K, H) + pre_s[:, None, :]
    t = _gelu(t).reshape(E_BLK, H).astype(jnp.bfloat16)
    t = jnp.dot(t, w2_ref[...], preferred_element_type=f32) + b2_ref[...]
    t = _gelu(t).astype(jnp.bfloat16)
    msg = jnp.dot(t, w3_ref[...], preferred_element_type=f32) + b3_ref[...]
    out_ref[0] = _ln(he + msg, n3g_ref[...], n3b_ref[...])


def _edge_update(h_E2, g2, hV2b, wp):
    grid = (B, L // R)
    full = lambda shape: pl.BlockSpec(shape, lambda b, i: (0,) * len(shape))
    in_specs = [
        pl.BlockSpec((1, E_BLK, H), lambda b, i: (b, i, 0)),
        pl.BlockSpec((1, E_BLK, H), lambda b, i: (b, i, 0)),
        pl.BlockSpec((1, R, H), lambda b, i: (b, i, 0)),
        full((H, H)), full((2 * H, H)), full((1, H)),
        full((H, H)), full((1, H)), full((H, H)), full((1, H)),
        full((1, H)), full((1, H)),
    ]
    return pl.pallas_call(
        _edge_body,
        grid=grid,
        in_specs=in_specs,
        out_specs=pl.BlockSpec((1, E_BLK, H), lambda b, i: (b, i, 0)),
        out_shape=jax.ShapeDtypeStruct((B, L * K, H), jnp.float32),
        name="tc_edge_update",
    )(h_E2, g2, hV2b, *wp)


def kernel(h_V, h_E, E_idx, mask_V, mask_attend, params):
    p = params
    bf = jnp.bfloat16
    f32 = jnp.float32

    h_E2 = h_E.reshape(B, L * K, H)
    idx_flat = (E_idx.astype(jnp.int32)
                + (jnp.arange(B, dtype=jnp.int32) * L)[:, None, None])
    idx_flat = idx_flat.reshape(TOTAL)
    mask_V3 = mask_V.reshape(B, L, 1)

    def wt(w):
        return jnp.transpose(w).astype(bf)

    def bias(b, n):
        return b.reshape(1, n).astype(f32)

    w1 = jnp.transpose(p['W1_w']).astype(bf)
    wp_node = (
        w1[:H], w1[H:], bias(p['W1_b'], H),
        wt(p['W2_w']), bias(p['W2_b'], H),
        wt(p['W3_w']), bias(p['W3_b'], H),
        bias(p['n1_g'], H), bias(p['n1_b'], H),
        wt(p['Win_w']), bias(p['Win_b'], 4 * H),
        wt(p['Wout_w']), bias(p['Wout_b'], H),
        bias(p['n2_g'], H), bias(p['n2_b'], H),
    )
    w11 = jnp.transpose(p['W11_w']).astype(bf)
    wp_edge = (
        w11[:H], w11[H:], bias(p['W11_b'], H),
        wt(p['W12_w']), bias(p['W12_b'], H),
        wt(p['W13_w']), bias(p['W13_b'], H),
        bias(p['n3_g'], H), bias(p['n3_b'], H),
    )

    g1 = _sc_gather(h_V.reshape(B * L, H), idx_flat)
    hV_new, hV_new_b = _node_update(
        h_V, h_E2, g1, mask_attend, mask_V3, wp_node)
    g2 = _sc_gather(hV_new.reshape(B * L, H), idx_flat)
    hE_new = _edge_update(h_E2, g2, hV_new_b, wp_edge)
    return (hV_new, hE_new.reshape(B, L, K, H))
